# baseline (speedup 1.0000x reference)
import functools

import jax
import jax.numpy as jnp
import numpy as np
from jax import lax
from jax.experimental import pallas as pl
from jax.experimental.pallas import tpu as pltpu
from jax.experimental.pallas import tpu_sc as plsc

N = 10000
E = 320000
D = 128

NC = 2
NS = 16
NW = NC * NS
CHUNK = 128
NCHUNKS = E // CHUNK
LANES = 16


def _mm_kernel(x_ref, wh_ref, wt_ref, h_ref, t_ref):
    xv = x_ref[...]
    for dst, w in ((h_ref, wh_ref), (t_ref, wt_ref)):
        y = 0.5 * jnp.dot(xv, w[...], preferred_element_type=jnp.float32)
        lo = jax.lax.bitcast_convert_type(
            y[:, :D // 2].astype(jnp.bfloat16), jnp.int16).astype(jnp.int32)
        hi = jax.lax.bitcast_convert_type(
            y[:, D // 2:].astype(jnp.bfloat16), jnp.int16).astype(jnp.int32)
        dst[...] = (lo & 0xFFFF) | (hi << 16)


def _node_transform(x, w_h, w_t):
    blk = 1000
    return pl.pallas_call(
        _mm_kernel,
        grid=(N // blk,),
        in_specs=[
            pl.BlockSpec((blk, D), lambda i: (i, 0)),
            pl.BlockSpec((D, D), lambda i: (0, 0)),
            pl.BlockSpec((D, D), lambda i: (0, 0)),
        ],
        out_specs=[
            pl.BlockSpec((blk, D // 2), lambda i: (i, 0)),
            pl.BlockSpec((blk, D // 2), lambda i: (i, 0)),
        ],
        out_shape=[
            jax.ShapeDtypeStruct((N, D // 2), jnp.int32),
            jax.ShapeDtypeStruct((N, D // 2), jnp.int32),
        ],
    )(x, w_h, w_t)


def _packed_perm():
    k = np.arange(D // 2)
    c_lo = 32 * (k // 16) + (k % 16)
    return np.concatenate([c_lo, c_lo + 16]).astype(np.int32)


NSLOT = 3
MAXC = -(-NCHUNKS // NW)
MAXC_R = -(-MAXC // NSLOT) * NSLOT


@functools.cache
def _make_edge_kernel():
    mesh = plsc.VectorSubcoreMesh(core_axis_name="c", subcore_axis_name="s")

    scratch = [
        pltpu.VMEM((NSLOT, CHUNK), jnp.int32),
        pltpu.VMEM((NSLOT, CHUNK), jnp.int32),
        pltpu.VMEM((NSLOT, CHUNK, D // 2), jnp.int32),
        pltpu.VMEM((NSLOT, CHUNK, D // 2), jnp.int32),
        pltpu.VMEM((NSLOT, CHUNK * D), jnp.float32),
    ] + [pltpu.SemaphoreType.DMA] * (4 * NSLOT)

    @functools.partial(
        pl.kernel,
        mesh=mesh,
        compiler_params=pltpu.CompilerParams(use_tc_tiling_on_sc=False),
        out_type=jax.ShapeDtypeStruct((E * D,), jnp.float32),
        scratch_types=scratch,
    )
    def _edge_kernel(h_hbm, t_hbm, row_hbm, col_hbm, out_hbm,
                     idxr, idxc, hb, tb, ob, *sems):
        SI = sems[0:NSLOT]
        SH = sems[NSLOT:2 * NSLOT]
        ST = sems[2 * NSLOT:3 * NSLOT]
        SO = sems[3 * NSLOT:4 * NSLOT]
        wid = lax.axis_index("s") * NC + lax.axis_index("c")
        n = (NCHUNKS - wid + NW - 1) // NW

        def cbase(c):
            return (wid + c * NW) * CHUNK

        def issue_idx(c, s):
            pltpu.async_copy(row_hbm.at[pl.ds(cbase(c), CHUNK)], idxr.at[s], SI[s])
            pltpu.async_copy(col_hbm.at[pl.ds(cbase(c), CHUNK)], idxc.at[s], SI[s])

        def wait_idx(s):
            pltpu.make_async_copy(row_hbm.at[pl.ds(0, CHUNK)], idxr.at[s], SI[s]).wait()
            pltpu.make_async_copy(col_hbm.at[pl.ds(0, CHUNK)], idxc.at[s], SI[s]).wait()

        def issue_gathers(s):
            pltpu.async_copy(h_hbm.at[idxr.at[s]], hb.at[s], SH[s])
            pltpu.async_copy(t_hbm.at[idxc.at[s]], tb.at[s], ST[s])

        def wait_gathers(s):
            pltpu.make_async_copy(h_hbm.at[idxr.at[s]], hb.at[s], SH[s]).wait()
            pltpu.make_async_copy(t_hbm.at[idxc.at[s]], tb.at[s], ST[s]).wait()

        def issue_writeout(c, s):
            pltpu.async_copy(ob.at[s], out_hbm.at[pl.ds(cbase(c) * D, CHUNK * D)], SO[s])

        def wait_writeout(s):
            pltpu.make_async_copy(ob.at[s], out_hbm.at[pl.ds(0, CHUNK * D)], SO[s]).wait()

        def compute(s):
            def row_body(r, z):
                himask = jnp.int32(-65536)
                rb = r * D
                for j in range(D // 32):
                    sl16 = pl.ds(j * LANES, LANES)
                    hw = hb[s, r, sl16]
                    tw = tb[s, r, sl16]
                    h_lo = lax.bitcast_convert_type(hw << 16, jnp.float32)
                    h_hi = lax.bitcast_convert_type(hw & himask, jnp.float32)
                    t_lo = lax.bitcast_convert_type(tw << 16, jnp.float32)
                    t_hi = lax.bitcast_convert_type(tw & himask, jnp.float32)
                    ob[s, pl.ds(rb + j * 32, LANES)] = jnp.maximum(h_lo + t_lo, 0.0)
                    ob[s, pl.ds(rb + j * 32 + LANES, LANES)] = jnp.maximum(h_hi + t_hi, 0.0)
                return z
            lax.fori_loop(0, CHUNK, row_body, 0, unroll=2)

        for c0 in range(NSLOT):
            issue_idx(c0, c0)
        for c0 in range(2):
            wait_idx(c0)
            issue_gathers(c0)

        def group_body(m, z):
            for k in range(NSLOT):
                c = m * NSLOT + k
                s = k

                @pl.when(c < n)
                def _():
                    wait_gathers(s)

                @pl.when(c + NSLOT < n)
                def _():
                    issue_idx(c + NSLOT, s)

                @pl.when(c + 2 < n)
                def _():
                    wait_idx((k + 2) % NSLOT)
                    issue_gathers((k + 2) % NSLOT)

                @pl.when((c >= NSLOT) & (c - NSLOT < n))
                def _():
                    wait_writeout(s)

                @pl.when(c < n)
                def _():
                    compute(s)
                    issue_writeout(c, s)
            return z

        lax.fori_loop(0, MAXC_R // NSLOT, group_body, 0)

        for x in range(MAXC_R - NSLOT, MAXC_R):
            @pl.when(x < n)
            def _():
                wait_writeout(x % NSLOT)

    return _edge_kernel


def kernel(x, edge_index, edge_attr, edge_type, w_self, w_h, w_t):
    del edge_attr, edge_type, w_self
    perm = jnp.asarray(_packed_perm())
    h, t = _node_transform(x, w_h[:, perm], w_t[:, perm])
    row = edge_index[0].astype(jnp.int32)
    col = edge_index[1].astype(jnp.int32)
    return _make_edge_kernel()(h, t, row, col).reshape(E, D)


# speedup vs baseline: 1.0307x; 1.0307x over previous
import functools

import jax
import jax.numpy as jnp
import numpy as np
from jax import lax
from jax.experimental import pallas as pl
from jax.experimental.pallas import tpu as pltpu
from jax.experimental.pallas import tpu_sc as plsc

N = 10000
E = 320000
D = 128

NC = 2
NS = 16
NW = NC * NS
CHUNK = 128
NCHUNKS = E // CHUNK
LANES = 16


def _mm_kernel(x_ref, wh_ref, wt_ref, h_ref, t_ref):
    xv = x_ref[...]
    for dst, w in ((h_ref, wh_ref), (t_ref, wt_ref)):
        y = 0.5 * jnp.dot(xv, w[...], preferred_element_type=jnp.float32)
        lo = jax.lax.bitcast_convert_type(
            y[:, :D // 2].astype(jnp.bfloat16), jnp.int16).astype(jnp.int32)
        hi = jax.lax.bitcast_convert_type(
            y[:, D // 2:].astype(jnp.bfloat16), jnp.int16).astype(jnp.int32)
        dst[...] = (lo & 0xFFFF) | (hi << 16)


def _node_transform(x, w_h, w_t):
    blk = 1000
    return pl.pallas_call(
        _mm_kernel,
        grid=(N // blk,),
        in_specs=[
            pl.BlockSpec((blk, D), lambda i: (i, 0)),
            pl.BlockSpec((D, D), lambda i: (0, 0)),
            pl.BlockSpec((D, D), lambda i: (0, 0)),
        ],
        out_specs=[
            pl.BlockSpec((blk, D // 2), lambda i: (i, 0)),
            pl.BlockSpec((blk, D // 2), lambda i: (i, 0)),
        ],
        out_shape=[
            jax.ShapeDtypeStruct((N, D // 2), jnp.int32),
            jax.ShapeDtypeStruct((N, D // 2), jnp.int32),
        ],
    )(x, w_h, w_t)


def _packed_perm():
    k = np.arange(D // 2)
    c_lo = 32 * (k // 16) + (k % 16)
    return np.concatenate([c_lo, c_lo + 16]).astype(np.int32)


NSLOT = 3
MAXC = -(-NCHUNKS // NW)
MAXC_R = -(-MAXC // NSLOT) * NSLOT


@functools.cache
def _make_edge_kernel():
    mesh = plsc.VectorSubcoreMesh(core_axis_name="c", subcore_axis_name="s")

    scratch = [
        pltpu.VMEM((NSLOT, CHUNK), jnp.int32),
        pltpu.VMEM((NSLOT, CHUNK), jnp.int32),
        pltpu.VMEM((NSLOT, CHUNK, D // 2), jnp.int32),
        pltpu.VMEM((NSLOT, CHUNK, D // 2), jnp.int32),
        pltpu.VMEM((NSLOT, CHUNK * D), jnp.float32),
    ] + [pltpu.SemaphoreType.DMA] * (4 * NSLOT)

    @functools.partial(
        pl.kernel,
        mesh=mesh,
        compiler_params=pltpu.CompilerParams(use_tc_tiling_on_sc=False),
        out_type=jax.ShapeDtypeStruct((E * D,), jnp.float32),
        scratch_types=scratch,
    )
    def _edge_kernel(h_hbm, t_hbm, ei_hbm, out_hbm,
                     idxr, idxc, hb, tb, ob, *sems):
        SI = sems[0:NSLOT]
        SH = sems[NSLOT:2 * NSLOT]
        ST = sems[2 * NSLOT:3 * NSLOT]
        SO = sems[3 * NSLOT:4 * NSLOT]
        wid = lax.axis_index("s") * NC + lax.axis_index("c")
        n = (NCHUNKS - wid + NW - 1) // NW

        def cbase(c):
            return (wid + c * NW) * CHUNK

        def issue_idx(c, s):
            pltpu.async_copy(ei_hbm.at[0, pl.ds(cbase(c), CHUNK)], idxr.at[s], SI[s])
            pltpu.async_copy(ei_hbm.at[1, pl.ds(cbase(c), CHUNK)], idxc.at[s], SI[s])

        def wait_idx(s):
            pltpu.make_async_copy(ei_hbm.at[0, pl.ds(0, CHUNK)], idxr.at[s], SI[s]).wait()
            pltpu.make_async_copy(ei_hbm.at[1, pl.ds(0, CHUNK)], idxc.at[s], SI[s]).wait()

        def issue_gathers(s):
            pltpu.async_copy(h_hbm.at[idxr.at[s]], hb.at[s], SH[s])
            pltpu.async_copy(t_hbm.at[idxc.at[s]], tb.at[s], ST[s])

        def wait_gathers(s):
            pltpu.make_async_copy(h_hbm.at[idxr.at[s]], hb.at[s], SH[s]).wait()
            pltpu.make_async_copy(t_hbm.at[idxc.at[s]], tb.at[s], ST[s]).wait()

        def issue_writeout(c, s):
            pltpu.async_copy(ob.at[s], out_hbm.at[pl.ds(cbase(c) * D, CHUNK * D)], SO[s])

        def wait_writeout(s):
            pltpu.make_async_copy(ob.at[s], out_hbm.at[pl.ds(0, CHUNK * D)], SO[s]).wait()

        def compute(s):
            def row_body(r, z):
                himask = jnp.int32(-65536)
                rb = r * D
                for j in range(D // 32):
                    sl16 = pl.ds(j * LANES, LANES)
                    hw = hb[s, r, sl16]
                    tw = tb[s, r, sl16]
                    h_lo = lax.bitcast_convert_type(hw << 16, jnp.float32)
                    h_hi = lax.bitcast_convert_type(hw & himask, jnp.float32)
                    t_lo = lax.bitcast_convert_type(tw << 16, jnp.float32)
                    t_hi = lax.bitcast_convert_type(tw & himask, jnp.float32)
                    ob[s, pl.ds(rb + j * 32, LANES)] = jnp.maximum(h_lo + t_lo, 0.0)
                    ob[s, pl.ds(rb + j * 32 + LANES, LANES)] = jnp.maximum(h_hi + t_hi, 0.0)
                return z
            lax.fori_loop(0, CHUNK, row_body, 0, unroll=2)

        for c0 in range(NSLOT):
            issue_idx(c0, c0)
        for c0 in range(2):
            wait_idx(c0)
            issue_gathers(c0)

        def group_body(m, z):
            for k in range(NSLOT):
                c = m * NSLOT + k
                s = k

                @pl.when(c < n)
                def _():
                    wait_gathers(s)

                @pl.when(c + NSLOT < n)
                def _():
                    issue_idx(c + NSLOT, s)

                @pl.when(c + 2 < n)
                def _():
                    wait_idx((k + 2) % NSLOT)
                    issue_gathers((k + 2) % NSLOT)

                @pl.when((c >= NSLOT) & (c - NSLOT < n))
                def _():
                    wait_writeout(s)

                @pl.when(c < n)
                def _():
                    compute(s)
                    issue_writeout(c, s)
            return z

        lax.fori_loop(0, MAXC_R // NSLOT, group_body, 0)

        for x in range(MAXC_R - NSLOT, MAXC_R):
            @pl.when(x < n)
            def _():
                wait_writeout(x % NSLOT)

    return _edge_kernel


def kernel(x, edge_index, edge_attr, edge_type, w_self, w_h, w_t):
    del edge_attr, edge_type, w_self
    perm = jnp.asarray(_packed_perm())
    h, t = _node_transform(x, w_h[:, perm], w_t[:, perm])
    return _make_edge_kernel()(h, t, edge_index.astype(jnp.int32)).reshape(E, D)


# speedup vs baseline: 1.0405x; 1.0095x over previous
import functools

import jax
import jax.numpy as jnp
import numpy as np
from jax import lax
from jax.experimental import pallas as pl
from jax.experimental.pallas import tpu as pltpu
from jax.experimental.pallas import tpu_sc as plsc

N = 10000
E = 320000
D = 128

NC = 2
NS = 16
NW = NC * NS
CHUNK = 128
NCHUNKS = E // CHUNK
LANES = 16


def _mm_kernel(x_ref, wh_ref, wt_ref, h_ref, t_ref):
    xv = x_ref[...]
    for dst, w in ((h_ref, wh_ref), (t_ref, wt_ref)):
        y = 0.5 * jnp.dot(xv, w[...], preferred_element_type=jnp.float32)
        lo = jax.lax.bitcast_convert_type(
            y[:, :D // 2].astype(jnp.bfloat16), jnp.int16).astype(jnp.int32)
        hi = jax.lax.bitcast_convert_type(
            y[:, D // 2:].astype(jnp.bfloat16), jnp.int16).astype(jnp.int32)
        dst[...] = (lo & 0xFFFF) | (hi << 16)


def _node_transform(x, w_h, w_t):
    blk = 1000
    return pl.pallas_call(
        _mm_kernel,
        grid=(N // blk,),
        in_specs=[
            pl.BlockSpec((blk, D), lambda i: (i, 0)),
            pl.BlockSpec((D, D), lambda i: (0, 0)),
            pl.BlockSpec((D, D), lambda i: (0, 0)),
        ],
        out_specs=[
            pl.BlockSpec((blk, D // 2), lambda i: (i, 0)),
            pl.BlockSpec((blk, D // 2), lambda i: (i, 0)),
        ],
        out_shape=[
            jax.ShapeDtypeStruct((N, D // 2), jnp.int32),
            jax.ShapeDtypeStruct((N, D // 2), jnp.int32),
        ],
    )(x, w_h, w_t)


def _packed_perm():
    k = np.arange(D // 2)
    c_lo = 32 * (k // 16) + (k % 16)
    return np.concatenate([c_lo, c_lo + 16]).astype(np.int32)


NSLOT = 3
EPW = E // NW
NFULL = EPW // CHUNK
TAIL = EPW - NFULL * CHUNK
assert NFULL % NSLOT == 0


@functools.cache
def _make_edge_kernel():
    mesh = plsc.VectorSubcoreMesh(core_axis_name="c", subcore_axis_name="s")

    scratch = [
        pltpu.VMEM((EPW,), jnp.int32),
        pltpu.VMEM((EPW,), jnp.int32),
        pltpu.VMEM((NSLOT, CHUNK, D // 2), jnp.int32),
        pltpu.VMEM((NSLOT, CHUNK, D // 2), jnp.int32),
        pltpu.VMEM((NSLOT, CHUNK * D), jnp.float32),
        pltpu.VMEM((TAIL, D // 2), jnp.int32),
        pltpu.VMEM((TAIL, D // 2), jnp.int32),
        pltpu.VMEM((TAIL * D,), jnp.float32),
    ] + [pltpu.SemaphoreType.DMA] * (3 * NSLOT + 3)

    @functools.partial(
        pl.kernel,
        mesh=mesh,
        compiler_params=pltpu.CompilerParams(use_tc_tiling_on_sc=False),
        out_type=jax.ShapeDtypeStruct((E * D,), jnp.float32),
        scratch_types=scratch,
    )
    def _edge_kernel(h_hbm, t_hbm, ei_hbm, out_hbm,
                     idxr, idxc, hb, tb, ob, hbt, tbt, obt, *sems):
        SH = sems[0:NSLOT]
        ST = sems[NSLOT:2 * NSLOT]
        SO = sems[2 * NSLOT:3 * NSLOT]
        SIDX, SHT, STT = sems[3 * NSLOT:]
        wid = lax.axis_index("s") * NC + lax.axis_index("c")
        base_w = wid * EPW

        cr = pltpu.async_copy(ei_hbm.at[0, pl.ds(base_w, EPW)], idxr, SIDX)
        cc = pltpu.async_copy(ei_hbm.at[1, pl.ds(base_w, EPW)], idxc, SIDX)
        cr.wait()
        cc.wait()

        def issue_gathers(c, s):
            pltpu.async_copy(h_hbm.at[idxr.at[pl.ds(c * CHUNK, CHUNK)]], hb.at[s], SH[s])
            pltpu.async_copy(t_hbm.at[idxc.at[pl.ds(c * CHUNK, CHUNK)]], tb.at[s], ST[s])

        def wait_gathers(s):
            pltpu.make_async_copy(h_hbm.at[idxr.at[pl.ds(0, CHUNK)]], hb.at[s], SH[s]).wait()
            pltpu.make_async_copy(t_hbm.at[idxc.at[pl.ds(0, CHUNK)]], tb.at[s], ST[s]).wait()

        def issue_writeout(c, s):
            pltpu.async_copy(
                ob.at[s], out_hbm.at[pl.ds((base_w + c * CHUNK) * D, CHUNK * D)], SO[s])

        def wait_writeout(s):
            pltpu.make_async_copy(ob.at[s], out_hbm.at[pl.ds(0, CHUNK * D)], SO[s]).wait()

        himask = jnp.int32(-65536)

        def body_row(hsrc, tsrc, osrc, r):
            rb = r * D
            for j in range(D // 32):
                sl16 = pl.ds(j * LANES, LANES)
                hw = hsrc[r, sl16]
                tw = tsrc[r, sl16]
                h_lo = lax.bitcast_convert_type(hw << 16, jnp.float32)
                h_hi = lax.bitcast_convert_type(hw & himask, jnp.float32)
                t_lo = lax.bitcast_convert_type(tw << 16, jnp.float32)
                t_hi = lax.bitcast_convert_type(tw & himask, jnp.float32)
                osrc[pl.ds(rb + j * 32, LANES)] = jnp.maximum(h_lo + t_lo, 0.0)
                osrc[pl.ds(rb + j * 32 + LANES, LANES)] = jnp.maximum(h_hi + t_hi, 0.0)

        def compute(s):
            def row_body(r, z):
                body_row(hb.at[s], tb.at[s], ob.at[s], r)
                return z
            lax.fori_loop(0, CHUNK, row_body, 0, unroll=2)

        pltpu.async_copy(h_hbm.at[idxr.at[pl.ds(NFULL * CHUNK, TAIL)]], hbt, SHT)
        pltpu.async_copy(t_hbm.at[idxc.at[pl.ds(NFULL * CHUNK, TAIL)]], tbt, STT)

        issue_gathers(0, 0)
        issue_gathers(1, 1)

        def group_body(m, z):
            for k in range(NSLOT):
                c = m * NSLOT + k
                s = k
                wait_gathers(s)

                @pl.when(c + 2 < NFULL)
                def _():
                    issue_gathers(c + 2, (k + 2) % NSLOT)

                @pl.when(c >= NSLOT)
                def _():
                    wait_writeout(s)

                compute(s)
                issue_writeout(c, s)
            return z

        lax.fori_loop(0, NFULL // NSLOT, group_body, 0)

        pltpu.make_async_copy(h_hbm.at[idxr.at[pl.ds(0, TAIL)]], hbt, SHT).wait()
        pltpu.make_async_copy(t_hbm.at[idxc.at[pl.ds(0, TAIL)]], tbt, STT).wait()

        def tail_body(r, z):
            body_row(hbt, tbt, obt, r)
            return z
        lax.fori_loop(0, TAIL, tail_body, 0)
        pltpu.sync_copy(obt, out_hbm.at[pl.ds((base_w + NFULL * CHUNK) * D, TAIL * D)])

        for s in range(NSLOT):
            wait_writeout(s)

    return _edge_kernel


def kernel(x, edge_index, edge_attr, edge_type, w_self, w_h, w_t):
    del edge_attr, edge_type, w_self
    perm = jnp.asarray(_packed_perm())
    h, t = _node_transform(x, w_h[:, perm], w_t[:, perm])
    return _make_edge_kernel()(h, t, edge_index.astype(jnp.int32)).reshape(E, D)


# speedup vs baseline: 1.0417x; 1.0012x over previous
import functools

import jax
import jax.numpy as jnp
import numpy as np
from jax import lax
from jax.experimental import pallas as pl
from jax.experimental.pallas import tpu as pltpu
from jax.experimental.pallas import tpu_sc as plsc

N = 10000
E = 320000
D = 128

NC = 2
NS = 16
NW = NC * NS
CHUNK = 128
LANES = 16


def _mm_kernel(x_ref, wh_ref, wt_ref, h_ref, t_ref):
    xv = x_ref[...]
    for dst, w in ((h_ref, wh_ref), (t_ref, wt_ref)):
        y = 0.5 * jnp.dot(xv, w[...], preferred_element_type=jnp.float32)
        lo = jax.lax.bitcast_convert_type(
            y[:, :D // 2].astype(jnp.bfloat16), jnp.int16).astype(jnp.int32)
        hi = jax.lax.bitcast_convert_type(
            y[:, D // 2:].astype(jnp.bfloat16), jnp.int16).astype(jnp.int32)
        dst[...] = (lo & 0xFFFF) | (hi << 16)


def _node_transform(x, w_h, w_t):
    blk = 1000
    return pl.pallas_call(
        _mm_kernel,
        grid=(N // blk,),
        in_specs=[
            pl.BlockSpec((blk, D), lambda i: (i, 0)),
            pl.BlockSpec((D, D), lambda i: (0, 0)),
            pl.BlockSpec((D, D), lambda i: (0, 0)),
        ],
        out_specs=[
            pl.BlockSpec((blk, D // 2), lambda i: (i, 0)),
            pl.BlockSpec((blk, D // 2), lambda i: (i, 0)),
        ],
        out_shape=[
            jax.ShapeDtypeStruct((N, D // 2), jnp.int32),
            jax.ShapeDtypeStruct((N, D // 2), jnp.int32),
        ],
    )(x, w_h, w_t)


def _packed_perm():
    k = np.arange(D // 2)
    c_lo = 32 * (k // 16) + (k % 16)
    return np.concatenate([c_lo, c_lo + 16]).astype(np.int32)


NSLOT = 3
EPW = E // NW
NFULL = EPW // CHUNK
TAIL = EPW - NFULL * CHUNK
assert NFULL % NSLOT == 0


@functools.cache
def _make_edge_kernel():
    mesh = plsc.VectorSubcoreMesh(core_axis_name="c", subcore_axis_name="s")

    scratch = [
        pltpu.VMEM((EPW,), jnp.int32),
        pltpu.VMEM((EPW,), jnp.int32),
        pltpu.VMEM((NSLOT, CHUNK, D // 2), jnp.int32),
        pltpu.VMEM((NSLOT, CHUNK, D // 2), jnp.int32),
        pltpu.VMEM((NSLOT, CHUNK * D), jnp.float32),
        pltpu.VMEM((TAIL, D // 2), jnp.int32),
        pltpu.VMEM((TAIL, D // 2), jnp.int32),
        pltpu.VMEM((TAIL * D,), jnp.float32),
    ] + [pltpu.SemaphoreType.DMA] * (3 * NSLOT + 3)

    @functools.partial(
        pl.kernel,
        mesh=mesh,
        compiler_params=pltpu.CompilerParams(use_tc_tiling_on_sc=False),
        out_type=jax.ShapeDtypeStruct((E * D,), jnp.float32),
        scratch_types=scratch,
    )
    def _edge_kernel(h_hbm, t_hbm, ei_hbm, out_hbm,
                     idxr, idxc, hb, tb, ob, hbt, tbt, obt, *sems):
        SH = sems[0:NSLOT]
        ST = sems[NSLOT:2 * NSLOT]
        SO = sems[2 * NSLOT:3 * NSLOT]
        SIDX, SHT, STT = sems[3 * NSLOT:]
        wid = lax.axis_index("s") * NC + lax.axis_index("c")
        base_w = wid * EPW

        cr = pltpu.async_copy(ei_hbm.at[0, pl.ds(base_w, EPW)], idxr, SIDX)
        cc = pltpu.async_copy(ei_hbm.at[1, pl.ds(base_w, EPW)], idxc, SIDX)
        cr.wait()
        cc.wait()

        def issue_gathers(c, s):
            pltpu.async_copy(h_hbm.at[idxr.at[pl.ds(c * CHUNK, CHUNK)]], hb.at[s], SH[s])
            pltpu.async_copy(t_hbm.at[idxc.at[pl.ds(c * CHUNK, CHUNK)]], tb.at[s], ST[s])

        def wait_gathers(s):
            pltpu.make_async_copy(h_hbm.at[idxr.at[pl.ds(0, CHUNK)]], hb.at[s], SH[s]).wait()
            pltpu.make_async_copy(t_hbm.at[idxc.at[pl.ds(0, CHUNK)]], tb.at[s], ST[s]).wait()

        def issue_writeout(c, s):
            pltpu.async_copy(
                ob.at[s], out_hbm.at[pl.ds((base_w + c * CHUNK) * D, CHUNK * D)], SO[s])

        def wait_writeout(s):
            pltpu.make_async_copy(ob.at[s], out_hbm.at[pl.ds(0, CHUNK * D)], SO[s]).wait()

        himask = jnp.int32(-65536)

        def body_row(hsrc, tsrc, osrc, r):
            rb = r * D
            for j in range(D // 32):
                sl16 = pl.ds(j * LANES, LANES)
                hw = hsrc[r, sl16]
                tw = tsrc[r, sl16]
                h_lo = lax.bitcast_convert_type(hw << 16, jnp.float32)
                h_hi = lax.bitcast_convert_type(hw & himask, jnp.float32)
                t_lo = lax.bitcast_convert_type(tw << 16, jnp.float32)
                t_hi = lax.bitcast_convert_type(tw & himask, jnp.float32)
                osrc[pl.ds(rb + j * 32, LANES)] = jnp.maximum(h_lo + t_lo, 0.0)
                osrc[pl.ds(rb + j * 32 + LANES, LANES)] = jnp.maximum(h_hi + t_hi, 0.0)

        def compute(s):
            def row_body(r, z):
                body_row(hb.at[s], tb.at[s], ob.at[s], r)
                return z
            lax.fori_loop(0, CHUNK, row_body, 0, unroll=2)

        pltpu.async_copy(h_hbm.at[idxr.at[pl.ds(NFULL * CHUNK, TAIL)]], hbt, SHT)
        pltpu.async_copy(t_hbm.at[idxc.at[pl.ds(NFULL * CHUNK, TAIL)]], tbt, STT)

        issue_gathers(0, 0)
        issue_gathers(1, 1)

        def group_body(m, z):
            for k in range(NSLOT):
                c = m * NSLOT + k
                s = k
                wait_gathers(s)

                @pl.when(c + 2 < NFULL)
                def _():
                    issue_gathers(c + 2, (k + 2) % NSLOT)

                @pl.when(c >= NSLOT)
                def _():
                    wait_writeout(s)

                compute(s)
                issue_writeout(c, s)
            return z

        lax.fori_loop(0, NFULL // NSLOT, group_body, 0)

        pltpu.make_async_copy(h_hbm.at[idxr.at[pl.ds(0, TAIL)]], hbt, SHT).wait()
        pltpu.make_async_copy(t_hbm.at[idxc.at[pl.ds(0, TAIL)]], tbt, STT).wait()

        def tail_body(r, z):
            body_row(hbt, tbt, obt, r)
            return z
        lax.fori_loop(0, TAIL, tail_body, 0)
        pltpu.sync_copy(obt, out_hbm.at[pl.ds((base_w + NFULL * CHUNK) * D, TAIL * D)])

        for s in range(NSLOT):
            wait_writeout(s)

    return _edge_kernel


def kernel(x, edge_index, edge_attr, edge_type, w_self, w_h, w_t):
    del edge_attr, edge_type, w_self
    perm = jnp.asarray(_packed_perm())
    h, t = _node_transform(x, w_h[:, perm], w_t[:, perm])
    return _make_edge_kernel()(h, t, edge_index.astype(jnp.int32)).reshape(E, D)
